# baseline (device time: 64337 ns/iter reference)
import jax
import jax.numpy as jnp
from jax import lax
from jax.experimental import pallas as pl
from jax.experimental.pallas import tpu as pltpu

N_DEV = 8
BLK = 1024


def kernel(x):
    m, n = x.shape
    nblk = m // BLK

    def body(x_ref, out_ref, xbuf, total_ref, slots_ref, load_sems,
             send_sems, recv_sems):
        my = lax.axis_index("i")

        barrier = pltpu.get_barrier_semaphore()
        for k in range(1, N_DEV):
            pl.semaphore_signal(
                barrier,
                inc=1,
                device_id=((my + k) % N_DEV,),
                device_id_type=pl.DeviceIdType.MESH,
            )
        pl.semaphore_wait(barrier, N_DEV - 1)

        def load(b):
            return pltpu.make_async_copy(
                x_ref.at[pl.ds(b * BLK, BLK), :],
                xbuf.at[b % 2],
                load_sems.at[b % 2],
            )

        tri = (
            lax.broadcasted_iota(jnp.int32, (BLK, BLK), 0)
            >= lax.broadcasted_iota(jnp.int32, (BLK, BLK), 1)
        ).astype(jnp.float32)

        load(0).start()
        carry = jnp.zeros((1, n), jnp.float32)
        for b in range(nblk):
            if b + 1 < nblk:
                load(b + 1).start()
            load(b).wait()
            cs = (
                jnp.dot(tri, xbuf[b % 2], preferred_element_type=jnp.float32)
                + carry
            )
            out_ref[pl.ds(b * BLK, BLK), :] = cs
            carry = cs[BLK - 1 : BLK, :]

        total_ref[...] = carry

        rdmas = []
        for k in range(1, N_DEV):
            rdma = pltpu.make_async_remote_copy(
                src_ref=total_ref,
                dst_ref=slots_ref.at[k - 1],
                send_sem=send_sems.at[k - 1],
                recv_sem=recv_sems.at[k - 1],
                device_id=((my + k) % N_DEV,),
                device_id_type=pl.DeviceIdType.MESH,
            )
            rdma.start()
            rdmas.append(rdma)
        for rdma in rdmas:
            rdma.wait()

        slots = slots_ref[...]
        ks = lax.broadcasted_iota(jnp.int32, (N_DEV - 1, 1, 1), 0) + 1
        mask = (ks <= my).astype(jnp.float32)
        prefix = jnp.sum(slots * mask, axis=0)

        def fix_step(b, _):
            sl = pl.ds(b * BLK, BLK)
            out_ref[sl, :] = out_ref[sl, :] + prefix
            return 0

        lax.fori_loop(0, nblk, fix_step, 0)

    return pl.pallas_call(
        body,
        out_shape=jax.ShapeDtypeStruct((m, n), jnp.float32),
        in_specs=[pl.BlockSpec(memory_space=pl.ANY)],
        out_specs=pl.BlockSpec(memory_space=pltpu.VMEM),
        scratch_shapes=[
            pltpu.VMEM((2, BLK, n), jnp.float32),
            pltpu.VMEM((1, n), jnp.float32),
            pltpu.VMEM((N_DEV - 1, 1, n), jnp.float32),
            pltpu.SemaphoreType.DMA((2,)),
            pltpu.SemaphoreType.DMA((N_DEV - 1,)),
            pltpu.SemaphoreType.DMA((N_DEV - 1,)),
        ],
        compiler_params=pltpu.CompilerParams(
            collective_id=0, vmem_limit_bytes=60 * 1024 * 1024
        ),
    )(x)


# device time: 60782 ns/iter; 1.0585x vs baseline; 1.0585x over previous
import jax
import jax.numpy as jnp
from jax import lax
from jax.experimental import pallas as pl
from jax.experimental.pallas import tpu as pltpu

N_DEV = 8
BLK = 512
import os
_PROBE = os.environ.get("PROBE", "")


def kernel(x):
    m, n = x.shape
    nblk = m // BLK

    def body(x_ref, out_ref, xbuf, total_ref, slots_ref, load_sems,
             send_sems, recv_sems):
        my = lax.axis_index("i")

        if _PROBE != "nocomm":
            barrier = pltpu.get_barrier_semaphore()
            for k in range(1, N_DEV):
                pl.semaphore_signal(
                    barrier,
                    inc=1,
                    device_id=((my + k) % N_DEV,),
                    device_id_type=pl.DeviceIdType.MESH,
                )
            pl.semaphore_wait(barrier, N_DEV - 1)

        def load(b):
            return pltpu.make_async_copy(
                x_ref.at[pl.ds(b * BLK, BLK), :],
                xbuf.at[b % 2],
                load_sems.at[b % 2],
            )

        tri = (
            lax.broadcasted_iota(jnp.int32, (BLK, BLK), 0)
            >= lax.broadcasted_iota(jnp.int32, (BLK, BLK), 1)
        ).astype(jnp.float32)

        load(0).start()
        carry = jnp.zeros((1, n), jnp.float32)
        for b in range(nblk):
            if b + 1 < nblk:
                load(b + 1).start()
            load(b).wait()
            if _PROBE == "nomatmul":
                cs = xbuf[b % 2] + carry
            else:
                cs = (
                    jnp.dot(tri, xbuf[b % 2], preferred_element_type=jnp.float32)
                    + carry
                )
            out_ref[pl.ds(b * BLK, BLK), :] = cs
            carry = cs[BLK - 1 : BLK, :]

        total_ref[...] = carry

        if _PROBE == "nocomm":
            slots_ref[...] = jnp.zeros_like(slots_ref)
        else:
            rdmas = []
            for k in range(1, N_DEV):
                rdma = pltpu.make_async_remote_copy(
                    src_ref=total_ref,
                    dst_ref=slots_ref.at[k - 1],
                    send_sem=send_sems.at[k - 1],
                    recv_sem=recv_sems.at[k - 1],
                    device_id=((my + k) % N_DEV,),
                    device_id_type=pl.DeviceIdType.MESH,
                )
                rdma.start()
                rdmas.append(rdma)
            for rdma in rdmas:
                rdma.wait()

        slots = slots_ref[...]
        ks = lax.broadcasted_iota(jnp.int32, (N_DEV - 1, 1, 1), 0) + 1
        mask = (ks <= my).astype(jnp.float32)
        prefix = jnp.sum(slots * mask, axis=0)

        if _PROBE != "nofixup":
            def fix_step(b, _):
                sl = pl.ds(b * BLK, BLK)
                out_ref[sl, :] = out_ref[sl, :] + prefix
                return 0

            lax.fori_loop(0, nblk, fix_step, 0)

    return pl.pallas_call(
        body,
        out_shape=jax.ShapeDtypeStruct((m, n), jnp.float32),
        in_specs=[pl.BlockSpec(memory_space=pl.ANY)],
        out_specs=pl.BlockSpec(memory_space=pltpu.VMEM),
        scratch_shapes=[
            pltpu.VMEM((2, BLK, n), jnp.float32),
            pltpu.VMEM((1, n), jnp.float32),
            pltpu.VMEM((N_DEV - 1, 1, n), jnp.float32),
            pltpu.SemaphoreType.DMA((2,)),
            pltpu.SemaphoreType.DMA((N_DEV - 1,)),
            pltpu.SemaphoreType.DMA((N_DEV - 1,)),
        ],
        compiler_params=pltpu.CompilerParams(
            collective_id=0, vmem_limit_bytes=60 * 1024 * 1024
        ),
    )(x)


# device time: 32685 ns/iter; 1.9684x vs baseline; 1.8596x over previous
import jax
import jax.numpy as jnp
from jax import lax
from jax.experimental import pallas as pl
from jax.experimental.pallas import tpu as pltpu

N_DEV = 8
BLK = 512
import os
_PROBE = os.environ.get("PROBE", "")


def kernel(x):
    m, n = x.shape
    nblk = m // BLK

    def body(x_ref, out_ref, xbuf, total_ref, slots_ref, load_sems,
             send_sems, recv_sems):
        my = lax.axis_index("i")

        if _PROBE != "nocomm":
            barrier = pltpu.get_barrier_semaphore()
            for k in range(1, N_DEV):
                pl.semaphore_signal(
                    barrier,
                    inc=1,
                    device_id=((my + k) % N_DEV,),
                    device_id_type=pl.DeviceIdType.MESH,
                )
            pl.semaphore_wait(barrier, N_DEV - 1)

        def load(b):
            return pltpu.make_async_copy(
                x_ref.at[pl.ds(b * BLK, BLK), :],
                xbuf.at[b % 2],
                load_sems.at[b % 2],
            )

        tri = (
            lax.broadcasted_iota(jnp.int32, (BLK, BLK), 0)
            >= lax.broadcasted_iota(jnp.int32, (BLK, BLK), 1)
        ).astype(jnp.float32)

        load(0).start()
        carry = jnp.zeros((1, n), jnp.float32)
        for b in range(nblk):
            if b + 1 < nblk:
                load(b + 1).start()
            load(b).wait()
            if _PROBE == "nomatmul":
                cs = xbuf[b % 2] + carry
            else:
                cs = (
                    jnp.dot(tri, xbuf[b % 2], preferred_element_type=jnp.float32)
                    + carry
                )
            out_ref[pl.ds(b * BLK, BLK), :] = cs
            carry = cs[BLK - 1 : BLK, :]

        total_ref[...] = carry

        if _PROBE == "nocomm":
            slots_ref[...] = jnp.zeros_like(slots_ref)
        else:
            rdmas = []
            for k in range(1, N_DEV):
                rdma = pltpu.make_async_remote_copy(
                    src_ref=total_ref,
                    dst_ref=slots_ref.at[k - 1],
                    send_sem=send_sems.at[k - 1],
                    recv_sem=recv_sems.at[k - 1],
                    device_id=((my + k) % N_DEV,),
                    device_id_type=pl.DeviceIdType.MESH,
                )
                rdma.start()
                rdmas.append(rdma)
            for rdma in rdmas:
                rdma.wait()

        slots = slots_ref[...]
        ks = lax.broadcasted_iota(jnp.int32, (N_DEV - 1, 1, 1), 0) + 1
        mask = (ks <= my).astype(jnp.float32)
        prefix = jnp.sum(slots * mask, axis=0)

        if _PROBE != "nofixup":
            def fix_step(b, _):
                sl = pl.ds(b * BLK, BLK)
                out_ref[sl, :] = out_ref[sl, :] + prefix
                return 0

            lax.fori_loop(0, nblk, fix_step, 0)

    return pl.pallas_call(
        body,
        out_shape=jax.ShapeDtypeStruct((m, n), jnp.float32),
        in_specs=[pl.BlockSpec(memory_space=pl.ANY)],
        out_specs=pl.BlockSpec(memory_space=pltpu.VMEM),
        scratch_shapes=[
            pltpu.VMEM((2, BLK, n), jnp.float32),
            pltpu.VMEM((1, n), jnp.float32),
            pltpu.VMEM((N_DEV - 1, 1, n), jnp.float32),
            pltpu.SemaphoreType.DMA((2,)),
            pltpu.SemaphoreType.DMA((N_DEV - 1,)),
            pltpu.SemaphoreType.DMA((N_DEV - 1,)),
        ],
        compiler_params=pltpu.CompilerParams(
            vmem_limit_bytes=60 * 1024 * 1024,
            **({} if _PROBE == "nocomm" else {"collective_id": 0}),
        ),
    )(x)
